# baseline (speedup 1.0000x reference)
import functools

import jax
import jax.numpy as jnp
from jax import lax
from jax.experimental import pallas as pl
from jax.experimental.pallas import tpu as pltpu
from jax.experimental.pallas import tpu_sc as plsc

M, K, D = 8, 8192, 256
N, H, W = 16, 32, 32

NC, NS = 2, 16
NW = NC * NS
LANES = 16

T = H * W
TOK = 128
ROWS = M * N * (T // TOK)
RPW = ROWS // NW
CPN = T // TOK


def _sc_decode(table, codes2):
    mesh = plsc.VectorSubcoreMesh(
        core_axis_name="c", subcore_axis_name="s", num_cores=NC,
        num_subcores=NS)

    @functools.partial(
        pl.kernel,
        mesh=mesh,
        out_type=jax.ShapeDtypeStruct((N, M * D, T), jnp.float32),
        scratch_types=[
            pltpu.VMEM((RPW, TOK), jnp.int32),
            pltpu.VMEM((TOK, D), jnp.float32),
            pltpu.VMEM((TOK, D), jnp.float32),
            pltpu.VMEM((D, TOK), jnp.float32),
            pltpu.SemaphoreType.DMA,
            pltpu.SemaphoreType.DMA,
            pltpu.SemaphoreType.DMA,
        ],
        compiler_params=pltpu.CompilerParams(
            needs_layout_passes=False, disable_bounds_checks=True),
    )
    def k(table_hbm, codes_hbm, out_hbm, codes_v, a0, a1, b_v,
          sa0, sa1, so):
        a_bufs = (a0, a1)
        sa = (sa0, sa1)

        wid = lax.axis_index("s") * NC + lax.axis_index("c")
        row0 = wid * RPW
        m_t = wid // (NW // M)
        iota = lax.iota(jnp.int32, LANES)

        pltpu.sync_copy(codes_hbm.at[pl.ds(row0, RPW)], codes_v)
        mk = jnp.broadcast_to(m_t * K, (LANES,))

        @plsc.parallel_loop(0, RPW, unroll=2)
        def _(r):
            for c in range(TOK // LANES):
                sl = pl.ds(c * LANES, LANES)
                codes_v[r, sl] = codes_v[r, sl] + mk

        pltpu.async_copy(table_hbm.at[codes_v.at[0]], a0, sa0)
        pltpu.async_copy(table_hbm.at[codes_v.at[1]], a1, sa1)

        lane_r = 8 * (iota // 8)
        c_lane = 128 * (iota % 8)

        def do_one(j, b):
            a_v = a_bufs[b]
            pltpu.make_async_copy(
                table_hbm.at[codes_v.at[0]], a_v, sa[b]).wait()

            jg = row0 + j
            n = (jg // CPN) % N
            t0 = (jg % CPN) * TOK
            dst = out_hbm.at[n, pl.ds(m_t * D, D), pl.ds(t0, TOK)]

            @pl.when(j >= 1)
            def _():
                pltpu.make_async_copy(b_v, dst, so).wait()

            for g in range(TOK // LANES):
                for h in range(D // 128):
                    r_vec = 16 * g + iota
                    c0 = jnp.broadcast_to(jnp.int32(h * 128), (LANES,))

                    @plsc.parallel_loop(0, 128, unroll=8, carry=c0)
                    def _(dd, c_vec):
                        x = plsc.load_gather(a_v, [r_vec, c_vec])
                        b_v[h * 128 + dd, pl.ds(g * LANES, LANES)] = x
                        return c_vec + 1

            pltpu.async_copy(b_v, dst, so)

            @pl.when(j + 2 < RPW)
            def _():
                pltpu.async_copy(
                    table_hbm.at[codes_v.at[j + 2]], a_v, sa[b])

        def pair(i, _):
            do_one(2 * i, 0)
            do_one(2 * i + 1, 1)
            return 0

        lax.fori_loop(0, RPW // 2, pair, 0)

        last = out_hbm.at[N - 1, pl.ds(m_t * D, D), pl.ds(T - TOK, TOK)]
        pltpu.make_async_copy(b_v, last, so).wait()

    return k(table, codes2)


def kernel(codes, codebook):
    table = codebook.reshape(M * K, D)
    codes2 = codes.transpose(3, 0, 1, 2).reshape(ROWS, TOK)
    out = _sc_decode(table, codes2)
    return out.reshape(N, M * D, H, W)


# speedup vs baseline: 1.0078x; 1.0078x over previous
import functools

import jax
import jax.numpy as jnp
from jax import lax
from jax.experimental import pallas as pl
from jax.experimental.pallas import tpu as pltpu
from jax.experimental.pallas import tpu_sc as plsc

M, K, D = 8, 8192, 256
N, H, W = 16, 32, 32

NC, NS = 2, 16
NW = NC * NS
LANES = 16

T = H * W
TOK = 128
ROWS = M * N * (T // TOK)
RPW = ROWS // NW
CPN = T // TOK
DH = D // 2
PAD = TOK


def _sc_decode(table, codes2):
    mesh = plsc.VectorSubcoreMesh(
        core_axis_name="c", subcore_axis_name="s", num_cores=NC,
        num_subcores=NS)

    @functools.partial(
        pl.kernel,
        mesh=mesh,
        out_type=jax.ShapeDtypeStruct((N, M * D, T), jnp.float32),
        scratch_types=[
            pltpu.VMEM((RPW, TOK), jnp.int32),
            pltpu.VMEM((TOK, D), jnp.float32),
            pltpu.VMEM((TOK, D), jnp.float32),
            pltpu.VMEM((DH, PAD), jnp.float32),
            pltpu.VMEM((DH, PAD), jnp.float32),
            pltpu.SemaphoreType.DMA,
            pltpu.SemaphoreType.DMA,
            pltpu.SemaphoreType.DMA,
            pltpu.SemaphoreType.DMA,
        ],
        compiler_params=pltpu.CompilerParams(
            needs_layout_passes=False, disable_bounds_checks=True),
    )
    def k(table_hbm, codes_hbm, out_hbm, codes_v, a0, a1, bh0, bh1,
          sa0, sa1, so0, so1):
        a_bufs = (a0, a1)
        b_half = (bh0, bh1)
        sa = (sa0, sa1)
        so = (so0, so1)

        wid = lax.axis_index("s") * NC + lax.axis_index("c")
        row0 = wid * RPW
        m_t = wid // (NW // M)
        iota = lax.iota(jnp.int32, LANES)

        pltpu.sync_copy(codes_hbm.at[pl.ds(row0, RPW)], codes_v)
        mk = jnp.broadcast_to(m_t * K, (LANES,))

        @plsc.parallel_loop(0, RPW, unroll=2)
        def _(r):
            for c in range(TOK // LANES):
                sl = pl.ds(c * LANES, LANES)
                codes_v[r, sl] = codes_v[r, sl] + mk

        pltpu.async_copy(table_hbm.at[codes_v.at[0]], a0, sa0)
        pltpu.async_copy(table_hbm.at[codes_v.at[1]], a1, sa1)

        czero = jnp.broadcast_to(jnp.int32(0), (LANES,))

        def do_one(j, b):
            a_v = a_bufs[b]
            pltpu.make_async_copy(
                table_hbm.at[codes_v.at[0]], a_v, sa[b]).wait()

            jg = row0 + j
            n = (jg // CPN) % N
            t0 = (jg % CPN) * TOK

            for h in range(2):
                b_v = b_half[h]
                dst = out_hbm.at[
                    n, pl.ds(m_t * D + h * DH, DH), pl.ds(t0, TOK)]
                src = b_v.at[:, pl.ds(0, TOK)]

                @pl.when(j >= 1)
                def _():
                    pltpu.make_async_copy(src, dst, so[h]).wait()

                for db in range(DH // LANES):
                    row_vec = db * LANES + iota

                    @plsc.parallel_loop(0, TOK, unroll=8, carry=czero)
                    def _(t, col_vec):
                        x = a_v[t, pl.ds(h * DH + db * LANES, LANES)]
                        plsc.store_scatter(b_v, [row_vec, col_vec], x)
                        return col_vec + 1

                pltpu.async_copy(src, dst, so[h])

            @pl.when(j + 2 < RPW)
            def _():
                pltpu.async_copy(
                    table_hbm.at[codes_v.at[j + 2]], a_v, sa[b])

        def pair(i, _):
            do_one(2 * i, 0)
            do_one(2 * i + 1, 1)
            return 0

        lax.fori_loop(0, RPW // 2, pair, 0)

        for h in range(2):
            last = out_hbm.at[
                N - 1, pl.ds(m_t * D + h * DH, DH), pl.ds(T - TOK, TOK)]
            pltpu.make_async_copy(
                b_half[h].at[:, pl.ds(0, TOK)], last, so[h]).wait()

    return k(table, codes2)


def kernel(codes, codebook):
    table = codebook.reshape(M * K, D)
    codes2 = codes.transpose(3, 0, 1, 2).reshape(ROWS, TOK)
    out = _sc_decode(table, codes2)
    return out.reshape(N, M * D, H, W)


# speedup vs baseline: 1.2200x; 1.2105x over previous
import functools

import jax
import jax.numpy as jnp
from jax import lax
from jax.experimental import pallas as pl
from jax.experimental.pallas import tpu as pltpu
from jax.experimental.pallas import tpu_sc as plsc

M, K, D = 8, 8192, 256
N, H, W = 16, 32, 32

NC, NS = 2, 16
NW = NC * NS
LANES = 16

T = H * W
B = N * T * M
ROWS = B // 128
ROWS_PER_W = ROWS // NW
CHUNK = 128


def _sc_gather(table, codes2):
    mesh = plsc.VectorSubcoreMesh(
        core_axis_name="c", subcore_axis_name="s", num_cores=NC,
        num_subcores=NS)

    @functools.partial(
        pl.kernel,
        mesh=mesh,
        out_type=jax.ShapeDtypeStruct((B, D), jnp.float32),
        scratch_types=[
            pltpu.VMEM((ROWS_PER_W, 128), jnp.int32),
            pltpu.VMEM((CHUNK, D), jnp.float32),
            pltpu.SemaphoreType.DMA,
        ],
    )
    def k(table_hbm, codes_hbm, out_hbm, idx_v, rows_v, sem):
        wid = lax.axis_index("s") * NC + lax.axis_index("c")
        row0 = wid * ROWS_PER_W
        pltpu.sync_copy(codes_hbm.at[pl.ds(row0, ROWS_PER_W)], idx_v)

        mv = (lax.iota(jnp.int32, LANES) % M) * K

        def add_m(j, _):
            for c in range(128 // LANES):
                sl = pl.ds(c * LANES, LANES)
                idx_v[j, sl] = idx_v[j, sl] + mv
            return 0

        lax.fori_loop(0, ROWS_PER_W, add_m, 0)

        def gather_chunk(j, _):
            pltpu.async_copy(table_hbm.at[idx_v.at[j]], rows_v, sem).wait()
            pltpu.sync_copy(
                rows_v, out_hbm.at[pl.ds((row0 + j) * 128, CHUNK)])
            return 0

        lax.fori_loop(0, ROWS_PER_W, gather_chunk, 0)

    return k(table, codes2)


def _tc_transpose(g3):
    def body(in_ref, out_ref):
        out_ref[...] = jnp.swapaxes(in_ref[...], 1, 2)

    return pl.pallas_call(
        body,
        grid=(N,),
        in_specs=[pl.BlockSpec((1, T, M * D), lambda n: (n, 0, 0))],
        out_specs=pl.BlockSpec((1, M * D, T), lambda n: (n, 0, 0)),
        out_shape=jax.ShapeDtypeStruct((N, M * D, T), jnp.float32),
    )(g3)


def kernel(codes, codebook):
    table = codebook.reshape(M * K, D)
    codes2 = codes.reshape(ROWS, 128)
    g = _sc_gather(table, codes2)
    out = _tc_transpose(g.reshape(N, T, M * D))
    return out.reshape(N, M * D, H, W)
